# Initial kernel scaffold; baseline (speedup 1.0000x reference)
import jax
import jax.numpy as jnp
from jax.experimental import pallas as pl


def kernel(z, t, pred_frac_coords, pred_atom_types, num_atoms, lengths, angles, edge_index, atom_emb, W_t1, b_t1, W_t2, b_t2, W_in, b_in, W_msg, b_msg, W_upd, b_upd, w_force, W_atom, b_atom):
    raise NotImplementedError("write your pallas kernel here")


# speedup vs baseline: 1.3165x; 1.3165x over previous
import functools
import math

import jax
import jax.numpy as jnp
from jax import lax
from jax.experimental import pallas as pl
from jax.experimental.pallas import tpu as pltpu
from jax.experimental.pallas import tpu_sc as plsc

N_CRYST = 400
A_PER = 25
N_ATOMS = N_CRYST * A_PER
NAP = 10240
N_EDGES = 320000
HID = 128
LAT = 256
TDIM = 128
MAXZ = 100
NRBF = 16
CUTOFF = 6.0

NCORES = 2
NSUB = 16
NW = NCORES * NSUB
EPT = N_EDGES // NW
CHUNK = 80
NCHUNK = EPT // CHUNK
GRP = CHUNK // 16
ROWS_PER_TILE = NAP // NSUB


def _silu(x):
    return x / (1.0 + jnp.exp(-x))


def _tc_te_lat(t_ref, len_ref, ang_ref, wt1_ref, bt1_ref, wt2_ref, bt2_ref,
               te_ref, lat9_ref):
    t = t_ref[:, 0:1]
    half = TDIM // 2
    k = lax.broadcasted_iota(jnp.int32, (N_CRYST, half), 1).astype(jnp.float32)
    freqs = jnp.exp(-math.log(10000.0) * k / (half - 1))
    args = t * freqs
    temb = jnp.concatenate([jnp.sin(args), jnp.cos(args)], axis=1)
    x = jnp.maximum(jnp.dot(temb, wt1_ref[:, :],
                            preferred_element_type=jnp.float32) + bt1_ref[0:1, :], 0.0)
    te_ref[:, :] = jnp.dot(x, wt2_ref[:, :],
                           preferred_element_type=jnp.float32) + bt2_ref[0:1, :]

    ang = ang_ref[:, 0:3] * (math.pi / 180.0)
    coss = jnp.cos(ang)
    a = len_ref[:, 0:1]
    b = len_ref[:, 1:2]
    c = len_ref[:, 2:3]
    ca = coss[:, 0:1]
    cb = coss[:, 1:2]
    cg = coss[:, 2:3]
    sg = jnp.clip(jnp.sin(ang[:, 2:3]), 1e-6, None)
    zz = jnp.zeros_like(a)
    cx = c * cb
    cy = c * (ca - cb * cg) / sg
    cz = jnp.sqrt(jnp.clip(c * c - cx * cx - cy * cy, 1e-6, None))
    lat9 = jnp.concatenate(
        [a, zz, zz, b * cg, b * sg, zz, cx, cy, cz,
         jnp.zeros((N_CRYST, 7), jnp.float32)], axis=1)
    lat9_ref[:, :] = lat9


def _tc_cart(fx_ref, fy_ref, fz_ref,
             l0_ref, l1_ref, l2_ref, l3_ref, l4_ref, l5_ref,
             l6_ref, l7_ref, l8_ref, cx_ref, cy_ref, cz_ref):
    fx, fy, fz = fx_ref[:, :], fy_ref[:, :], fz_ref[:, :]
    cx_ref[:, :] = fx * l0_ref[:, :] + fy * l3_ref[:, :] + fz * l6_ref[:, :]
    cy_ref[:, :] = fx * l1_ref[:, :] + fy * l4_ref[:, :] + fz * l7_ref[:, :]
    cz_ref[:, :] = fx * l2_ref[:, :] + fy * l5_ref[:, :] + fz * l8_ref[:, :]


def _sc_dvec(cx_hbm, cy_hbm, cz_hbm, src_hbm, dst_hbm,
             dx_hbm, dy_hbm, dz_hbm,
             cxv, cyv, czv, idxs_v, idxd_v, dxb, dyb, dzb):
    cid = lax.axis_index("c")
    sid = lax.axis_index("s")
    wid = sid * NCORES + cid
    pltpu.sync_copy(cx_hbm, cxv)
    pltpu.sync_copy(cy_hbm, cyv)
    pltpu.sync_copy(cz_hbm, czv)

    def chunk_body(ch, _):
        base = wid * EPT + ch * CHUNK
        pltpu.sync_copy(src_hbm.at[pl.ds(base, CHUNK)], idxs_v)
        pltpu.sync_copy(dst_hbm.at[pl.ds(base, CHUNK)], idxd_v)
        for g in range(GRP):
            sl = pl.ds(16 * g, 16)
            sv = idxs_v[sl]
            dv = idxd_v[sl]
            dxb[sl] = plsc.load_gather(cxv, [sv]) - plsc.load_gather(cxv, [dv])
            dyb[sl] = plsc.load_gather(cyv, [sv]) - plsc.load_gather(cyv, [dv])
            dzb[sl] = plsc.load_gather(czv, [sv]) - plsc.load_gather(czv, [dv])
        pltpu.sync_copy(dxb, dx_hbm.at[pl.ds(base, CHUNK)])
        pltpu.sync_copy(dyb, dy_hbm.at[pl.ds(base, CHUNK)])
        pltpu.sync_copy(dzb, dz_hbm.at[pl.ds(base, CHUNK)])
        return 0
    lax.fori_loop(0, NCHUNK, chunk_body, 0)


def _tc_geom_soa(dx_ref, dy_ref, dz_ref, ux_ref, uy_ref, uz_ref, dist_ref):
    dx, dy, dz = dx_ref[:, :], dy_ref[:, :], dz_ref[:, :]
    dist = jnp.sqrt(dx * dx + dy * dy + dz * dz + 1e-8)
    dist_ref[:, :] = dist
    ux_ref[:, :] = dx / dist
    uy_ref[:, :] = dy / dist
    uz_ref[:, :] = dz / dist


def _tc_rterm(dist_ref, wr0_ref, wr1_ref, r0_ref, r1_ref):
    d = dist_ref[0]
    rows = lax.broadcasted_iota(jnp.int32, (512, 512), 0)
    cols = lax.broadcasted_iota(jnp.int32, (512, 512), 1)
    eye = (rows == cols).astype(jnp.float32)
    dcol = lax.dot_general(eye, d, (((1,), (1,)), ((), ())),
                           preferred_element_type=jnp.float32)
    k = lax.broadcasted_iota(jnp.int32, (512, NRBF), 1).astype(jnp.float32)
    centers = k * (CUTOFF / (NRBF - 1))
    rbf = jnp.exp(-((dcol - centers) ** 2) * 2.0)
    r0_ref[:, :] = jnp.dot(rbf, wr0_ref[:, :], preferred_element_type=jnp.float32)
    r1_ref[:, :] = jnp.dot(rbf, wr1_ref[:, :], preferred_element_type=jnp.float32)


def _tc_hinit(types_ref, zb_ref, teb_ref, emb_ref, wh_ref, wz_ref, wte_ref,
              bin_ref, ws_ref, bmsg_ref, wd_ref, h_ref, a_ref, b_ref):
    ids = types_ref[:, 0:1] - 1
    lanes = lax.broadcasted_iota(jnp.int32, (ids.shape[0], 128), 1)
    oh = (ids == lanes).astype(jnp.float32)
    h0 = jnp.dot(oh, emb_ref[:, :], preferred_element_type=jnp.float32)
    x = (jnp.dot(h0, wh_ref[:, :], preferred_element_type=jnp.float32)
         + jnp.dot(zb_ref[:, :], wz_ref[:, :], preferred_element_type=jnp.float32)
         + jnp.dot(teb_ref[:, :], wte_ref[:, :], preferred_element_type=jnp.float32)
         + bin_ref[0:1, :])
    h = _silu(x)
    h_ref[:, :] = h
    a_ref[:, :] = jnp.dot(h, ws_ref[:, :], preferred_element_type=jnp.float32) + bmsg_ref[0:1, :]
    b_ref[:, :] = jnp.dot(h, wd_ref[:, :], preferred_element_type=jnp.float32)


def _sc_msg(a_hbm, b_hbm, r_hbm, ux_hbm, uy_hbm, uz_hbm, src_hbm, dst_hbm,
            wf_hbm, zagg_hbm, zf_hbm,
            agg_out, fx_out, fy_out, fz_out,
            agg_sh, fsh_x, fsh_y, fsh_z,
            idxs_v, idxd_v, av, bv, rv, uxv, uyv, uzv, mbuf,
            fxb, fyb, fzb, wfv):
    cid = lax.axis_index("c")
    sid = lax.axis_index("s")
    wid = sid * NCORES + cid

    @pl.when(sid == 0)
    def _init():
        pltpu.sync_copy(zagg_hbm, agg_sh)
        pltpu.sync_copy(zf_hbm, fsh_x)
        pltpu.sync_copy(zf_hbm, fsh_y)
        pltpu.sync_copy(zf_hbm, fsh_z)

    pltpu.sync_copy(wf_hbm, wfv)
    plsc.subcore_barrier()

    lane = lax.broadcasted_iota(jnp.int32, (16,), 0)

    def chunk_body(ch, _):
        base = wid * EPT + ch * CHUNK
        pltpu.sync_copy(src_hbm.at[pl.ds(base, CHUNK)], idxs_v)
        pltpu.sync_copy(dst_hbm.at[pl.ds(base, CHUNK)], idxd_v)
        pltpu.sync_copy(a_hbm.at[idxs_v], av)
        pltpu.sync_copy(b_hbm.at[idxd_v], bv)
        pltpu.sync_copy(r_hbm.at[pl.ds(base, CHUNK)], rv)
        pltpu.sync_copy(ux_hbm.at[pl.ds(base, CHUNK)], uxv)
        pltpu.sync_copy(uy_hbm.at[pl.ds(base, CHUNK)], uyv)
        pltpu.sync_copy(uz_hbm.at[pl.ds(base, CHUNK)], uzv)

        for g in range(GRP):
            gsl = pl.ds(16 * g, 16)

            def e_body(e, fsv):
                row = 16 * g + e
                acc = jnp.zeros((16,), jnp.float32)
                for j in range(HID // 16):
                    sl = pl.ds(16 * j, 16)
                    x = av[row, sl] + bv[row, sl] + rv[row, sl]
                    m = x / (1.0 + jnp.exp(-x))
                    mbuf[row, sl] = m
                    acc = acc + m * wfv[sl]
                fs = jnp.sum(acc)
                return jnp.where(lane == e, fs, fsv)

            fsv = lax.fori_loop(0, 16, e_body, jnp.zeros((16,), jnp.float32))
            fxb[gsl] = fsv * uxv[gsl]
            fyb[gsl] = fsv * uyv[gsl]
            fzb[gsl] = fsv * uzv[gsl]
        pltpu.sync_copy(mbuf, agg_sh.at[idxd_v], add=True)
        pltpu.sync_copy(fxb, fsh_x.at[idxd_v], add=True)
        pltpu.sync_copy(fyb, fsh_y.at[idxd_v], add=True)
        pltpu.sync_copy(fzb, fsh_z.at[idxd_v], add=True)
        return 0
    lax.fori_loop(0, NCHUNK, chunk_body, 0)

    plsc.subcore_barrier()
    r0 = sid * ROWS_PER_TILE
    out_base = cid * NAP + r0
    pltpu.sync_copy(agg_sh.at[pl.ds(r0, ROWS_PER_TILE)],
                    agg_out.at[pl.ds(out_base, ROWS_PER_TILE)])
    pltpu.sync_copy(fsh_x.at[pl.ds(r0, ROWS_PER_TILE)],
                    fx_out.at[pl.ds(out_base, ROWS_PER_TILE)])
    pltpu.sync_copy(fsh_y.at[pl.ds(r0, ROWS_PER_TILE)],
                    fy_out.at[pl.ds(out_base, ROWS_PER_TILE)])
    pltpu.sync_copy(fsh_z.at[pl.ds(r0, ROWS_PER_TILE)],
                    fz_out.at[pl.ds(out_base, ROWS_PER_TILE)])


def _tc_update(agg0_ref, agg1_ref, h_ref, wu_ref, bu_ref, ws_ref, bmsg_ref,
               wd_ref, h1_ref, a_ref, b_ref):
    agg = agg0_ref[:, :] + agg1_ref[:, :]
    u = _silu(jnp.dot(agg, wu_ref[:, :], preferred_element_type=jnp.float32)
              + bu_ref[0:1, :])
    h1 = h_ref[:, :] + u
    h1_ref[:, :] = h1
    a_ref[:, :] = jnp.dot(h1, ws_ref[:, :], preferred_element_type=jnp.float32) + bmsg_ref[0:1, :]
    b_ref[:, :] = jnp.dot(h1, wd_ref[:, :], preferred_element_type=jnp.float32)


def _tc_final(agg0_ref, agg1_ref, h_ref, wu_ref, bu_ref, wa_ref, ba_ref,
              logits_ref):
    agg = agg0_ref[:, :] + agg1_ref[:, :]
    u = _silu(jnp.dot(agg, wu_ref[:, :], preferred_element_type=jnp.float32)
              + bu_ref[0:1, :])
    h2 = h_ref[:, :] + u
    logits_ref[:, :] = jnp.dot(h2, wa_ref[:, :], preferred_element_type=jnp.float32) + ba_ref[0:1, :]


def _tc_fsum(f0a_ref, f0b_ref, f1a_ref, f1b_ref, out_ref):
    out_ref[:, :] = (f0a_ref[:, :] + f0b_ref[:, :]
                     + f1a_ref[:, :] + f1b_ref[:, :])


_MESH = None


def _sc_mesh():
    global _MESH
    if _MESH is None:
        _MESH = plsc.VectorSubcoreMesh(core_axis_name="c", subcore_axis_name="s")
    return _MESH


@jax.jit
def kernel(z, t, pred_frac_coords, pred_atom_types, num_atoms, lengths, angles,
           edge_index, atom_emb, W_t1, b_t1, W_t2, b_t2, W_in, b_in, W_msg,
           b_msg, W_upd, b_upd, w_force, W_atom, b_atom):
    f32 = jnp.float32
    src = edge_index[0].astype(jnp.int32)
    dst = edge_index[1].astype(jnp.int32)

    t2 = t.reshape(N_CRYST, 1)
    len_p = jnp.pad(lengths, ((0, 0), (0, 13)))
    ang_p = jnp.pad(angles, ((0, 0), (0, 13)))
    bt1 = b_t1.reshape(1, 4 * TDIM)
    bt2 = b_t2.reshape(1, TDIM)

    te, lat9 = pl.pallas_call(
        _tc_te_lat,
        out_shape=(jax.ShapeDtypeStruct((N_CRYST, TDIM), f32),
                   jax.ShapeDtypeStruct((N_CRYST, 16), f32)),
    )(t2, len_p, ang_p, W_t1, bt1, W_t2, bt2)

    def bcast(x):
        n, d = x.shape
        out = jnp.broadcast_to(x[:, None, :], (n, A_PER, d)).reshape(n * A_PER, d)
        return jnp.pad(out, ((0, NAP - N_ATOMS), (0, 0)))

    def plane(v):
        return jnp.pad(v, (0, NAP - N_ATOMS)).reshape(NAP // 128, 128)

    z_b = bcast(z)
    te_b = bcast(te)
    lat9_b = bcast(lat9)
    lplanes = [lat9_b[:, i].reshape(NAP // 128, 128) for i in range(9)]
    fplanes = [plane(pred_frac_coords[:, i]) for i in range(3)]
    types_p = jnp.pad(pred_atom_types.astype(jnp.int32), (0, NAP - N_ATOMS),
                      constant_values=1).reshape(NAP, 1)

    ap_spec = pl.BlockSpec((NAP // 128, 128), lambda: (0, 0))
    cxp, cyp, czp = pl.pallas_call(
        _tc_cart,
        in_specs=[ap_spec] * 12,
        out_specs=[ap_spec] * 3,
        out_shape=(jax.ShapeDtypeStruct((NAP // 128, 128), f32),) * 3,
    )(*fplanes, *lplanes)
    cx1, cy1, cz1 = (cxp.reshape(NAP), cyp.reshape(NAP), czp.reshape(NAP))

    dx, dy, dz = pl.kernel(
        _sc_dvec,
        out_type=(jax.ShapeDtypeStruct((N_EDGES,), f32),) * 3,
        mesh=_sc_mesh(),
        compiler_params=pltpu.CompilerParams(needs_layout_passes=False),
        scratch_types=[
            pltpu.VMEM((NAP,), f32),
            pltpu.VMEM((NAP,), f32),
            pltpu.VMEM((NAP,), f32),
            pltpu.VMEM((CHUNK,), jnp.int32),
            pltpu.VMEM((CHUNK,), jnp.int32),
            pltpu.VMEM((CHUNK,), f32),
            pltpu.VMEM((CHUNK,), f32),
            pltpu.VMEM((CHUNK,), f32),
        ],
    )(cx1, cy1, cz1, src, dst)

    e_plane = (N_EDGES // 128, 128)
    ux, uy, uz, dist = pl.pallas_call(
        _tc_geom_soa,
        out_shape=(jax.ShapeDtypeStruct(e_plane, f32),) * 4,
    )(dx.reshape(e_plane), dy.reshape(e_plane), dz.reshape(e_plane))

    Wr0 = W_msg[0, 2 * HID:, :]
    Wr1 = W_msg[1, 2 * HID:, :]
    grid_e = N_EDGES // 512
    dist_r = dist.reshape(grid_e, 1, 512)
    R0, R1 = pl.pallas_call(
        _tc_rterm,
        grid=(grid_e,),
        in_specs=[pl.BlockSpec((1, 1, 512), lambda i: (i, 0, 0)),
                  pl.BlockSpec((NRBF, HID), lambda i: (0, 0)),
                  pl.BlockSpec((NRBF, HID), lambda i: (0, 0))],
        out_specs=[pl.BlockSpec((512, HID), lambda i: (i, 0)),
                   pl.BlockSpec((512, HID), lambda i: (i, 0))],
        out_shape=(jax.ShapeDtypeStruct((N_EDGES, HID), f32),
                   jax.ShapeDtypeStruct((N_EDGES, HID), f32)),
    )(dist_r, Wr0, Wr1)

    emb_p = jnp.pad(atom_emb, ((0, 128 - MAXZ), (0, 0)))
    Wh = W_in[:HID, :]
    Wz = W_in[HID:HID + LAT, :]
    Wte = W_in[HID + LAT:, :]
    bin2 = b_in.reshape(1, HID)
    bmsg0 = b_msg[0].reshape(1, HID)
    bmsg1 = b_msg[1].reshape(1, HID)
    Ws0, Wd0 = W_msg[0, :HID, :], W_msg[0, HID:2 * HID, :]
    Ws1, Wd1 = W_msg[1, :HID, :], W_msg[1, HID:2 * HID, :]

    full = lambda r, c: pl.BlockSpec((r, c), lambda i: (0, 0))
    grid_a = NAP // 512
    h, A0, B0 = pl.pallas_call(
        _tc_hinit,
        grid=(grid_a,),
        in_specs=[pl.BlockSpec((512, 1), lambda i: (i, 0)),
                  pl.BlockSpec((512, LAT), lambda i: (i, 0)),
                  pl.BlockSpec((512, TDIM), lambda i: (i, 0)),
                  full(128, HID), full(HID, HID), full(LAT, HID),
                  full(TDIM, HID), full(1, HID), full(HID, HID),
                  full(1, HID), full(HID, HID)],
        out_specs=[pl.BlockSpec((512, HID), lambda i: (i, 0))] * 3,
        out_shape=(jax.ShapeDtypeStruct((NAP, HID), f32),) * 3,
    )(types_p, z_b, te_b, emb_p, Wh, Wz, Wte, bin2, Ws0, bmsg0, Wd0)

    zagg = jnp.zeros((NAP, HID), f32)
    zf = jnp.zeros((NAP,), f32)
    wf = w_force.astype(f32)
    ux1, uy1, uz1 = ux.reshape(N_EDGES), uy.reshape(N_EDGES), uz.reshape(N_EDGES)

    def sc_message(A, B, R):
        return pl.kernel(
            _sc_msg,
            out_type=(jax.ShapeDtypeStruct((NCORES * NAP, HID), f32),
                      jax.ShapeDtypeStruct((NCORES * NAP,), f32),
                      jax.ShapeDtypeStruct((NCORES * NAP,), f32),
                      jax.ShapeDtypeStruct((NCORES * NAP,), f32)),
            mesh=_sc_mesh(),
            compiler_params=pltpu.CompilerParams(needs_layout_passes=False),
            scratch_types=[
                pltpu.VMEM_SHARED((NAP, HID), f32),
                pltpu.VMEM_SHARED((NAP,), f32),
                pltpu.VMEM_SHARED((NAP,), f32),
                pltpu.VMEM_SHARED((NAP,), f32),
                pltpu.VMEM((CHUNK,), jnp.int32),
                pltpu.VMEM((CHUNK,), jnp.int32),
                pltpu.VMEM((CHUNK, HID), f32),
                pltpu.VMEM((CHUNK, HID), f32),
                pltpu.VMEM((CHUNK, HID), f32),
                pltpu.VMEM((CHUNK,), f32),
                pltpu.VMEM((CHUNK,), f32),
                pltpu.VMEM((CHUNK,), f32),
                pltpu.VMEM((CHUNK, HID), f32),
                pltpu.VMEM((CHUNK,), f32),
                pltpu.VMEM((CHUNK,), f32),
                pltpu.VMEM((CHUNK,), f32),
                pltpu.VMEM((HID,), f32),
            ],
        )(A, B, R, ux1, uy1, uz1, src, dst, wf, zagg, zf)

    agg0cat, f0x, f0y, f0z = sc_message(A0, B0, R0)

    bu0 = b_upd[0].reshape(1, HID)
    bu1 = b_upd[1].reshape(1, HID)
    h1, A1, B1 = pl.pallas_call(
        _tc_update,
        grid=(grid_a,),
        in_specs=[pl.BlockSpec((512, HID), lambda i: (i, 0))] * 3 + [
            full(HID, HID), full(1, HID), full(HID, HID),
            full(1, HID), full(HID, HID)],
        out_specs=[pl.BlockSpec((512, HID), lambda i: (i, 0))] * 3,
        out_shape=(jax.ShapeDtypeStruct((NAP, HID), f32),) * 3,
    )(agg0cat[:NAP], agg0cat[NAP:], h, W_upd[0], bu0, Ws1, bmsg1, Wd1)

    agg1cat, f1x, f1y, f1z = sc_message(A1, B1, R1)

    Wa_p = jnp.pad(W_atom, ((0, 0), (0, 128 - MAXZ)))
    ba_p = jnp.pad(b_atom, (0, 128 - MAXZ)).reshape(1, 128)
    logits_p = pl.pallas_call(
        _tc_final,
        grid=(grid_a,),
        in_specs=[pl.BlockSpec((512, HID), lambda i: (i, 0))] * 3 + [
            full(HID, HID), full(1, HID), full(HID, 128), full(1, 128)],
        out_specs=pl.BlockSpec((512, 128), lambda i: (i, 0)),
        out_shape=jax.ShapeDtypeStruct((NAP, 128), f32),
    )(agg1cat[:NAP], agg1cat[NAP:], h1, W_upd[1], bu1, Wa_p, ba_p)

    def fsum2(b0, b1):
        shp = (NAP // 128, 128)
        return pl.pallas_call(
            _tc_fsum,
            in_specs=[ap_spec] * 4,
            out_specs=ap_spec,
            out_shape=jax.ShapeDtypeStruct(shp, f32),
        )(b0[:NAP].reshape(shp), b0[NAP:].reshape(shp),
          b1[:NAP].reshape(shp), b1[NAP:].reshape(shp))

    fx = fsum2(f0x, f1x).reshape(NAP)[:N_ATOMS]
    fy = fsum2(f0y, f1y).reshape(NAP)[:N_ATOMS]
    fz = fsum2(f0z, f1z).reshape(NAP)[:N_ATOMS]

    force = jnp.stack([fx, fy, fz], axis=-1)
    logits = logits_p[:N_ATOMS, :MAXZ]
    return (force, logits)
